# baseline (speedup 1.0000x reference)
import functools

import jax
import jax.numpy as jnp
from jax import lax
from jax.experimental import pallas as pl
from jax.experimental.pallas import tpu as pltpu
from jax.experimental.pallas import tpu_sc as plsc

_B, _L, _W, _V = 4, 2048, 2048, 2048
_ROWS = _B * _L
_NC, _NS = 2, 16
_NW = _NC * _NS
_RPW = _ROWS // _NW
_CHUNK = 16
_NCHUNK = _RPW // _CHUNK


def _build():
    mesh = plsc.VectorSubcoreMesh(core_axis_name="c", subcore_axis_name="s")

    @functools.partial(
        pl.kernel,
        mesh=mesh,
        out_type=jax.ShapeDtypeStruct((_ROWS, _W), jnp.float32),
        scratch_types=[
            pltpu.VMEM((_RPW,), jnp.int32),
            pltpu.VMEM((_RPW,), jnp.int32),
            pltpu.VMEM((_NCHUNK, _CHUNK), jnp.int32),
            pltpu.VMEM((2, _CHUNK, _W), jnp.float32),
            pltpu.SemaphoreType.DMA,
            pltpu.SemaphoreType.DMA,
            pltpu.SemaphoreType.DMA,
            pltpu.SemaphoreType.DMA,
        ],
    )
    def k(table_hbm, idx_hbm, mask_hbm, out_hbm,
          idx_v, mask_v, eff_v, rows_v, g0, g1, p0, p1):
        gsem = (g0, g1)
        psem = (p0, p1)
        wid = lax.axis_index("s") * _NC + lax.axis_index("c")
        base = wid * _RPW

        pltpu.sync_copy(idx_hbm.at[pl.ds(base, _RPW)], idx_v)
        pltpu.sync_copy(mask_hbm.at[pl.ds(base, _RPW)], mask_v)

        for g in range(_NCHUNK):
            i = idx_v[pl.ds(g * _CHUNK, _CHUNK)]
            m = mask_v[pl.ds(g * _CHUNK, _CHUNK)]
            eff_v[g, pl.ds(0, _CHUNK)] = jnp.where(m != 0, i, _V)

        dg = [None] * _NCHUNK
        dp = [None] * _NCHUNK
        dg[0] = pltpu.async_copy(
            table_hbm.at[eff_v.at[0]], rows_v.at[0], gsem[0])
        for c in range(_NCHUNK):
            b = c % 2
            dg[c].wait()
            dp[c] = pltpu.async_copy(
                rows_v.at[b], out_hbm.at[pl.ds(base + c * _CHUNK, _CHUNK)],
                psem[b])
            if c + 1 < _NCHUNK:
                b2 = (c + 1) % 2
                if c >= 1:
                    dp[c - 1].wait()
                dg[c + 1] = pltpu.async_copy(
                    table_hbm.at[eff_v.at[c + 1]], rows_v.at[b2], gsem[b2])
        dp[_NCHUNK - 2].wait()
        dp[_NCHUNK - 1].wait()

    return k


_K = _build()


def kernel(obs_pos, obs_mask, table):
    idx = obs_pos.reshape(_ROWS).astype(jnp.int32)
    mask = obs_mask.reshape(_ROWS).astype(jnp.int32)
    table_z = jnp.concatenate(
        [table, jnp.zeros((1, _W), jnp.float32)], axis=0)
    out = _K(table_z, idx, mask)
    return out.reshape(_B, _L, _W)
